# Initial kernel scaffold; baseline (speedup 1.0000x reference)
import jax
import jax.numpy as jnp
from jax.experimental import pallas as pl


def kernel(x, edge_index, W_self_0, W_neigh_0, b_0, W_self_1, W_neigh_1, b_1, W_self_2, W_neigh_2, b_2):
    raise NotImplementedError("write your pallas kernel here")


# speedup vs baseline: 3.8968x; 3.8968x over previous
import functools

import jax
import jax.numpy as jnp
from jax import lax
from jax.experimental import pallas as pl
from jax.experimental.pallas import tpu as pltpu
from jax.experimental.pallas import tpu_sc as plsc

N_NODES = 10000
N_EDGES = 320000
FDIM = 128

NC = 2
NS = 16
NW = NC * NS
CHUNK = 128
E_PAD = ((N_EDGES + NW * CHUNK - 1) // (NW * CHUNK)) * (NW * CHUNK)
EPW = E_PAD // NW
CPW = EPW // CHUNK
ROWS_PT = (N_NODES // NS) // 8 * 8
ROWS_TAIL = N_NODES - ROWS_PT * NS
ACC_ROWS = N_NODES + 8

_MESH = plsc.VectorSubcoreMesh(core_axis_name="c", subcore_axis_name="s")


def _init_acc(acc, sid, rows):
    r0 = sid * ROWS_PT
    nfull = ROWS_PT // CHUNK
    rem = ROWS_PT % CHUNK
    for j in range(nfull):
        pltpu.sync_copy(rows, acc.at[pl.ds(r0 + j * CHUNK, CHUNK)])
    if rem:
        pltpu.sync_copy(
            rows.at[pl.ds(0, rem)], acc.at[pl.ds(r0 + nfull * CHUNK, rem)]
        )

    @pl.when(sid == NS - 1)
    def _():
        pltpu.sync_copy(
            rows.at[pl.ds(0, ROWS_TAIL)],
            acc.at[pl.ds(ROWS_PT * NS, ROWS_TAIL)],
        )


def _copy_out(acc, cid, sid, out):
    r0 = sid * ROWS_PT
    o0 = pl.multiple_of(cid * N_NODES + r0, 8)
    pltpu.sync_copy(acc.at[pl.ds(r0, ROWS_PT)], out.at[pl.ds(o0, ROWS_PT)])
    ot = pl.multiple_of(cid * N_NODES + ROWS_PT * NS, 8)

    @pl.when(sid == NS - 1)
    def _():
        pltpu.sync_copy(
            acc.at[pl.ds(ROWS_PT * NS, ROWS_TAIL)], out.at[pl.ds(ot, ROWS_TAIL)]
        )


def _seg_sum_body(src_hbm, dst_hbm, h_hbm, zeros_hbm, out_s, sidx, didx, rows,
                  acc_s, sem):
    cid = lax.axis_index("c")
    sid = lax.axis_index("s")
    wid = cid * NS + sid

    pltpu.sync_copy(zeros_hbm, rows)
    _init_acc(acc_s, sid, rows)
    plsc.subcore_barrier()

    base = wid * EPW

    def chunk_body(j, _):
        off = pl.multiple_of(base + j * CHUNK, CHUNK)
        pltpu.sync_copy(src_hbm.at[pl.ds(off, CHUNK)], sidx)
        pltpu.sync_copy(dst_hbm.at[pl.ds(off, CHUNK)], didx)
        pltpu.async_copy(h_hbm.at[sidx], rows, sem).wait()
        pltpu.sync_copy(rows, acc_s.at[didx], add=True)
        return 0

    lax.fori_loop(0, CPW, chunk_body, 0)
    plsc.subcore_barrier()
    _copy_out(acc_s, cid, sid, out_s)


_seg_sum = pl.kernel(
    _seg_sum_body,
    out_type=jax.ShapeDtypeStruct((NC * N_NODES, FDIM), jnp.float32),
    mesh=_MESH,
    scratch_types=[
        pltpu.VMEM((CHUNK,), jnp.int32),
        pltpu.VMEM((CHUNK,), jnp.int32),
        pltpu.VMEM((CHUNK, FDIM), jnp.float32),
        pltpu.VMEM_SHARED((ACC_ROWS, FDIM), jnp.float32),
        pltpu.SemaphoreType.DMA,
    ],
)


def _counts_body(dst_hbm, zeros_hbm, ones_hbm, out_c, didx, rows, obuf, acc_c):
    cid = lax.axis_index("c")
    sid = lax.axis_index("s")
    wid = cid * NS + sid

    pltpu.sync_copy(zeros_hbm, rows)
    pltpu.sync_copy(ones_hbm, obuf)
    _init_acc(acc_c, sid, rows)
    plsc.subcore_barrier()

    base = wid * EPW

    def chunk_body(j, _):
        off = pl.multiple_of(base + j * CHUNK, CHUNK)
        pltpu.sync_copy(dst_hbm.at[pl.ds(off, CHUNK)], didx)
        pltpu.sync_copy(obuf, acc_c.at[didx], add=True)
        return 0

    lax.fori_loop(0, CPW, chunk_body, 0)
    plsc.subcore_barrier()
    _copy_out(acc_c, cid, sid, out_c)


_counts = pl.kernel(
    _counts_body,
    out_type=jax.ShapeDtypeStruct((NC * N_NODES, FDIM), jnp.float32),
    mesh=_MESH,
    scratch_types=[
        pltpu.VMEM((CHUNK,), jnp.int32),
        pltpu.VMEM((CHUNK, FDIM), jnp.float32),
        pltpu.VMEM((CHUNK, FDIM), jnp.float32),
        pltpu.VMEM_SHARED((ACC_ROWS, FDIM), jnp.float32),
    ],
)

_BLK = 2000


def _combine_body(relu, s_ref, c_ref, h_ref, wn_ref, ws_ref, b_ref, o_ref):
    s = s_ref[0] + s_ref[1]
    cnt = c_ref[0][:, :1] + c_ref[1][:, :1]
    agg = s / jnp.maximum(cnt, 1.0)
    y = jnp.dot(agg, wn_ref[...], preferred_element_type=jnp.float32)
    y = y + jnp.dot(h_ref[...], ws_ref[...], preferred_element_type=jnp.float32)
    y = y + b_ref[...]
    if relu:
        y = jnp.maximum(y, 0.0)
    o_ref[...] = y


def _make_combine(relu):
    return pl.pallas_call(
        functools.partial(_combine_body, relu),
        grid=(N_NODES // _BLK,),
        in_specs=[
            pl.BlockSpec((NC, _BLK, FDIM), lambda i: (0, i, 0)),
            pl.BlockSpec((NC, _BLK, FDIM), lambda i: (0, i, 0)),
            pl.BlockSpec((_BLK, FDIM), lambda i: (i, 0)),
            pl.BlockSpec((FDIM, FDIM), lambda i: (0, 0)),
            pl.BlockSpec((FDIM, FDIM), lambda i: (0, 0)),
            pl.BlockSpec((1, FDIM), lambda i: (0, 0)),
        ],
        out_specs=pl.BlockSpec((_BLK, FDIM), lambda i: (i, 0)),
        out_shape=jax.ShapeDtypeStruct((N_NODES, FDIM), jnp.float32),
    )


_combine_relu = _make_combine(True)
_combine_last = _make_combine(False)


def kernel(x, edge_index, W_self_0, W_neigh_0, b_0, W_self_1, W_neigh_1, b_1,
           W_self_2, W_neigh_2, b_2):
    src = edge_index[0]
    dst = edge_index[1]
    pad = E_PAD - N_EDGES
    src_p = jnp.concatenate([src, jnp.zeros((pad,), jnp.int32)])
    dst_p = jnp.concatenate([dst, jnp.full((pad,), N_NODES, jnp.int32)])
    zeros = jnp.zeros((CHUNK, FDIM), jnp.float32)
    ones = jnp.ones((CHUNK, FDIM), jnp.float32)

    cnt = _counts(dst_p, zeros, ones).reshape(NC, N_NODES, FDIM)

    params = [
        (W_self_0, W_neigh_0, b_0),
        (W_self_1, W_neigh_1, b_1),
        (W_self_2, W_neigh_2, b_2),
    ]
    h = x
    for i, (Ws, Wn, b) in enumerate(params):
        S = _seg_sum(src_p, dst_p, h, zeros)
        combine = _combine_relu if i < 2 else _combine_last
        h = combine(
            S.reshape(NC, N_NODES, FDIM), cnt, h, Wn, Ws, b.reshape(1, FDIM)
        )
    return h
